# Initial kernel scaffold; baseline (speedup 1.0000x reference)
import jax
import jax.numpy as jnp
from jax.experimental import pallas as pl


def kernel(buffer, spikes, delays, ptr):
    raise NotImplementedError("write your pallas kernel here")


# speedup vs baseline: 1.0172x; 1.0172x over previous
import functools

import jax
import jax.numpy as jnp
from jax import lax
from jax.experimental import pallas as pl
from jax.experimental.pallas import tpu as pltpu
from jax.experimental.pallas import tpu_sc as plsc

NC = 2
NS = 16
L = 16
NW = NC * NS

G = 128
K = 8


@functools.lru_cache(maxsize=None)
def _build(T, N):
    C = N // NW
    NB = C // (G * K)

    mesh = plsc.VectorSubcoreMesh(core_axis_name="c", subcore_axis_name="s")

    @functools.partial(
        pl.kernel,
        out_type=jax.ShapeDtypeStruct((N,), jnp.float32),
        mesh=mesh,
        scratch_types=[
            pltpu.VMEM((C,), jnp.int32),
            pltpu.VMEM((C,), jnp.float32),
            pltpu.VMEM((C,), jnp.int32),
            pltpu.VMEM((C,), jnp.float32),
            pltpu.VMEM((L,), jnp.int32),
            pltpu.SemaphoreType.DMA,
        ],
    )
    def sc_kernel(flat_hbm, spikes_hbm, delays_hbm, ptr_hbm, out_hbm,
                  d_v, s_v, i_v, g_v, p_v, sem):
        wid = lax.axis_index("s") * NC + lax.axis_index("c")
        base = wid * C
        pltpu.sync_copy(delays_hbm.at[pl.ds(base, C)], d_v)
        pltpu.sync_copy(spikes_hbm.at[pl.ds(base, C)], s_v)
        pltpu.sync_copy(ptr_hbm, p_v)
        pvec = p_v[...]
        iot = lax.iota(jnp.int32, L)

        def cbody(j, _):
            off = j * L
            d = d_v[pl.ds(off, L)]
            r = (pvec + T - d) & (T - 1)
            i_v[pl.ds(off, L)] = r * N + (base + off) + iot
            return 0

        lax.fori_loop(0, C // L, cbody, 0, unroll=4)

        def gbody(b, _):
            copies = []
            for t in range(K):
                off = (b * K + t) * G
                copies.append(pltpu.async_copy(
                    flat_hbm.at[i_v.at[pl.ds(off, G)]],
                    g_v.at[pl.ds(off, G)], sem))
            for cp in copies:
                cp.wait()
            return 0

        lax.fori_loop(0, NB, gbody, 0)

        def sbody(j, _):
            off = j * L
            d = d_v[pl.ds(off, L)]
            g_v[pl.ds(off, L)] = jnp.where(d == 0, s_v[pl.ds(off, L)],
                                           g_v[pl.ds(off, L)])
            return 0

        lax.fori_loop(0, C // L, sbody, 0, unroll=4)
        pltpu.sync_copy(g_v, out_hbm.at[pl.ds(base, C)])

    return sc_kernel


def kernel(buffer, spikes, delays, ptr):
    T, N = buffer.shape
    flat = buffer.reshape(T * N)
    ptr_v = jnp.full((L,), ptr, dtype=jnp.int32)
    return _build(T, N)(flat, spikes.astype(jnp.float32),
                        delays.astype(jnp.int32), ptr_v)


# speedup vs baseline: 1.7820x; 1.7519x over previous
import functools

import jax
import jax.numpy as jnp
from jax import lax
from jax.experimental import pallas as pl
from jax.experimental.pallas import tpu as pltpu
from jax.experimental.pallas import tpu_sc as plsc

NC = 2
NS = 16
L = 16
NW = NC * NS

W = 256


@functools.lru_cache(maxsize=None)
def _build(T, N):
    C = N // NW
    NSLAB = C // W

    mesh = plsc.VectorSubcoreMesh(core_axis_name="c", subcore_axis_name="s")

    @functools.partial(
        pl.kernel,
        out_type=jax.ShapeDtypeStruct((N,), jnp.float32),
        mesh=mesh,
        compiler_params=pltpu.CompilerParams(needs_layout_passes=False),
        scratch_types=[
            pltpu.VMEM((C,), jnp.int32),
            pltpu.VMEM((C,), jnp.float32),
            pltpu.VMEM((C,), jnp.float32),
            pltpu.VMEM((2, T, W), jnp.float32),
            pltpu.VMEM((L,), jnp.int32),
            pltpu.SemaphoreType.DMA,
            pltpu.SemaphoreType.DMA,
        ],
    )
    def sc_kernel(buf_hbm, spikes_hbm, delays_hbm, ptr_hbm, out_hbm,
                  d_v, s_v, o_v, slab_v, p_v, sem0, sem1):
        wid = lax.axis_index("s") * NC + lax.axis_index("c")
        base = wid * C
        pltpu.sync_copy(delays_hbm.at[pl.ds(base, C)], d_v)
        pltpu.sync_copy(spikes_hbm.at[pl.ds(base, C)], s_v)
        pltpu.sync_copy(ptr_hbm, p_v)
        pvec = p_v[...]
        iot = lax.iota(jnp.int32, L)

        def fire(b, p, sem):
            pltpu.async_copy(
                buf_hbm.at[:, pl.ds(base + b * W, W)],
                slab_v.at[p], sem)

        def drain(p, sem):
            pltpu.make_async_copy(
                buf_hbm.at[:, pl.ds(base, W)], slab_v.at[p], sem).wait()

        def extract(b, p):
            slab = slab_v.at[p]
            for j in range(W // L):
                off = b * W + j * L
                d = d_v[pl.ds(off, L)]
                r = (pvec + T - d) & (T - 1)
                g = plsc.load_gather(slab, [r, j * L + iot])
                o_v[pl.ds(off, L)] = jnp.where(d == 0, s_v[pl.ds(off, L)], g)

        fire(0, 0, sem0)

        def body(k, _):
            b0 = 2 * k
            b1 = b0 + 1
            fire(b1, 1, sem1)
            drain(0, sem0)
            extract(b0, 0)

            @pl.when(b1 + 1 < NSLAB)
            def _():
                fire(b1 + 1, 0, sem0)

            drain(1, sem1)
            extract(b1, 1)
            return 0

        lax.fori_loop(0, NSLAB // 2, body, 0)
        pltpu.sync_copy(o_v, out_hbm.at[pl.ds(base, C)])

    return sc_kernel


def kernel(buffer, spikes, delays, ptr):
    T, N = buffer.shape
    ptr_v = jnp.full((L,), ptr, dtype=jnp.int32)
    return _build(T, N)(buffer, spikes.astype(jnp.float32),
                        delays.astype(jnp.int32), ptr_v)
